# baseline (speedup 1.0000x reference)
import jax
import jax.numpy as jnp
from jax.experimental import pallas as pl
from jax.experimental.pallas import tpu as pltpu

_BF16 = jnp.bfloat16

_IR_ARCH = [
    (3, 1, 1, 16),
    (3, 2, 6, 24), (3, 1, 3, 24), (3, 1, 6, 24), (3, 1, 6, 24),
    (5, 2, 6, 32), (5, 1, 3, 32), (5, 1, 6, 32), (3, 1, 6, 32),
    (5, 2, 6, 64), (5, 1, 3, 64), (5, 1, 6, 64), (5, 1, 6, 64),
    (5, 1, 6, 112), (5, 1, 6, 112), (5, 1, 1, 112), (5, 1, 6, 112),
    (5, 2, 6, 184), (5, 1, 6, 184), (5, 1, 6, 184), (5, 1, 6, 184),
    (3, 1, 6, 352),
]

_STAGES = [[1], [2, 3, 4, 5], [6, 7, 8, 9],
           [10, 11, 12, 13, 14, 15, 16, 17], [18, 19, 20, 21, 22]]
_PITCHES = [128, None, None, None, None]


def _build_geoms(h_in):
    geoms = {}
    cin = 16
    h = h_in
    for idx, (k, s, e, cout) in enumerate(_IR_ARCH, start=1):
        pad = k // 2
        hs = h // s
        ws = hs
        n_phase = s * s
        if s == 1:
            pb = pa = pad
        else:
            pb, pa = (pad + 1) // 2, pad // 2
        m = hs * ws
        band = (hs + pb + pa) * ws
        has_expand = e != 1
        cmid = cin * e if has_expand else cin
        geoms[idx] = dict(
            k=k, s=s, pad=pad, ws=ws, m=m, pb=pb, pa=pa, n_phase=n_phase,
            cin=cin, cmid=cmid, cout=cout,
            has_expand=has_expand, use_res=(s == 1 and cin == cout))
        cin = cout
        h = hs
    return geoms


def _run_block(cur, it, exp_ref, g, pg, is_head):
    k, s, pad = g["k"], g["s"], g["pad"]
    ws, pb = g["ws"], g["pb"]
    n_phase, cmid = g["n_phase"], g["cmid"]
    m_ = g["m"]
    hs = m_ // ws
    hw, wsp, prows, band = pg["hw"], pg["wsp"], pg["prows"], pg["band"]
    if g["has_expand"]:
        w_exp = next(it)
        b_exp = next(it)
    w_dw = next(it)
    b_dw = next(it)
    w_proj = next(it)
    b_proj = next(it)

    pitched = pg["mode"] == "pitched"
    if g["has_expand"]:
        e = jnp.dot(cur, w_exp[...], preferred_element_type=jnp.float32)
        e = jnp.maximum(e + b_exp[...], 0.0)
    else:
        e = cur.astype(jnp.float32)

    if pitched:
        @pl.when(pl.program_id(0) == 0)
        def _zero():
            exp_ref[...] = jnp.zeros(exp_ref.shape, exp_ref.dtype)
    else:
        zero_ranges = [(0, ws + pb * ws)]
        for ph in range(1, n_phase):
            zero_ranges.append((ws + (ph - 1) * band + pb * ws + m_,
                                ws + ph * band + pb * ws))
        zero_ranges.append((ws + (n_phase - 1) * band + pb * ws + m_,
                            pg["rows"]))
        for a, b in zero_ranges:
            exp_ref[a:b, :] = jnp.zeros((b - a, cmid), jnp.float32)

    if pitched:
        if s == 1:
            for i in range(hs):
                dst = hw + (pb + i) * wsp + hw
                src = i * wsp + hw if not is_head else i * ws
                exp_ref[dst:dst + ws, :] = e[src:src + ws, :]
        else:
            for ph in range(n_phase):
                py, px = ph // s, ph % s
                base = hw + ph * band
                for i in range(hs):
                    dst = base + (pb + i) * wsp + hw
                    exp_ref[dst:dst + ws, :] = (
                        e[(s * i + py) * ws:(s * i + py + 1) * ws,
                          px * cmid:(px + 1) * cmid])
    else:
        if s == 1:
            off = ws + pb * ws
            exp_ref[off:off + m_, :] = e
        else:
            for ph in range(n_phase):
                py, px = ph // s, ph % s
                off = ws + ph * band + pb * ws
                for i in range(hs):
                    exp_ref[off + i * ws:off + (i + 1) * ws, :] = (
                        e[(s * i + py) * ws:(s * i + py + 1) * ws,
                          px * cmid:(px + 1) * cmid])

    col = None
    masks = {}
    accs = [None, None]
    t = 0
    for ky in range(k):
        dy = ky - pad
        py, my = dy % s, dy // s
        for kx in range(k):
            dx = kx - pad
            px, mx = dx % s, dx // s
            if pitched:
                start = hw + (py * s + px) * band + (pb + my) * wsp + mx
                tap = exp_ref[start:start + prows, :]
            else:
                start = ws + (py * s + px) * band + (pb + my) * ws + mx
                tap = exp_ref[start:start + m_, :]
                if mx != 0:
                    if col is None:
                        col = jax.lax.broadcasted_iota(
                            jnp.int32, (m_, 1), 0) % ws
                    if mx not in masks:
                        masks[mx] = (col < ws - mx) if mx > 0 else (col >= -mx)
                    tap = jnp.where(masks[mx], tap, 0)
            contrib = tap * w_dw[t]
            slot = t & 1
            accs[slot] = contrib if accs[slot] is None else accs[slot] + contrib
            t += 1
    dw = accs[0] if accs[1] is None else accs[0] + accs[1]
    dw = jnp.maximum(dw + b_dw[...], 0.0)

    y = jnp.dot(dw.astype(_BF16), w_proj[...],
                preferred_element_type=jnp.float32)
    y = y + b_proj[...]
    if g["use_res"]:
        if is_head and pitched:
            y = y + exp_ref[hw + pb * wsp:hw + pb * wsp + prows, :]
        else:
            y = y + cur.astype(jnp.float32)
    return y.astype(_BF16)


def _stage_call(x2d, stage_geoms, stage_params, stem_wb=None, pitch_wsp=None):
    n = x2d.shape[0]
    n_params = (2 if stem_wb is not None else 0) + sum(len(p) for p in stage_params)
    with_stem = stem_wb is not None

    hl = 8
    pgs = []
    for g in stage_geoms:
        hs = g["m"] // g["ws"]
        if pitch_wsp is not None:
            band = (g["pb"] + hs + g["pa"]) * pitch_wsp
            pgs.append(dict(mode="pitched", hw=hl, wsp=pitch_wsp,
                            prows=hs * pitch_wsp, band=band,
                            rows=g["n_phase"] * band + 2 * hl))
        else:
            ws = g["ws"]
            band = (g["pb"] + hs + g["pa"]) * ws
            pgs.append(dict(mode="masked", hw=0, wsp=ws, prows=g["m"],
                            band=band, rows=g["n_phase"] * band + 2 * ws))

    def body(*refs):
        x_ref = refs[0]
        o_ref = refs[1 + n_params]
        scratches = refs[2 + n_params:]
        it = iter(refs[1:1 + n_params])
        cur = x_ref[...]
        if with_stem:
            w0 = next(it)
            b0 = next(it)
            y0 = jnp.dot(cur, w0[...], preferred_element_type=jnp.float32)
            cur = jnp.maximum(y0 + b0[...], 0.0).astype(_BF16)
        for bi, (g, pg, exp_ref) in enumerate(zip(stage_geoms, pgs, scratches)):
            cur = _run_block(cur, it, exp_ref, g, pg, is_head=(bi == 0))
        gl, pgl = stage_geoms[-1], pgs[-1]
        if pgl["mode"] == "pitched":
            ws_l = gl["ws"]
            for i in range(gl["m"] // ws_l):
                o_ref[i * ws_l:(i + 1) * ws_l, :] = (
                    cur[i * pgl["wsp"] + hl:i * pgl["wsp"] + hl + ws_l, :])
        else:
            o_ref[...] = cur

    def _const_spec(arr):
        nd = arr.ndim
        return pl.BlockSpec(arr.shape, lambda i, _nd=nd: (0,) * _nd)

    in_specs = [pl.BlockSpec((None,) + x2d.shape[1:], lambda i: (i, 0, 0))]
    operands = [x2d]
    if with_stem:
        for arr in stem_wb:
            in_specs.append(_const_spec(arr))
            operands.append(arr)
    for plist in stage_params:
        for arr in plist:
            in_specs.append(_const_spec(arr))
            operands.append(arr)

    g_last = stage_geoms[-1]
    out = pl.pallas_call(
        body,
        grid=(n,),
        in_specs=in_specs,
        out_specs=pl.BlockSpec((None, g_last["m"], g_last["cout"]),
                               lambda i: (i, 0, 0)),
        out_shape=jax.ShapeDtypeStruct((n, g_last["m"], g_last["cout"]), _BF16),
        scratch_shapes=[pltpu.VMEM((pg["rows"], g["cmid"]), jnp.float32)
                        for g, pg in zip(stage_geoms, pgs)],
        compiler_params=pltpu.CompilerParams(
            dimension_semantics=("parallel",)),
    )(*operands)
    return out


def _stem_patches(x_nchw, w0):
    x = jnp.transpose(x_nchw, (0, 2, 3, 1)).astype(_BF16)
    nb, h, w, c = x.shape
    ho, wo = h // 2, w // 2
    xpad = jnp.pad(x, ((0, 0), (1, 1), (1, 1), (0, 0)))
    taps = [xpad[:, ky:ky + 2 * (ho - 1) + 1:2, kx:kx + 2 * (wo - 1) + 1:2, :]
            for ky in range(3) for kx in range(3)]
    patches = jnp.stack(taps, axis=3).reshape(nb, ho * wo, 9 * c)
    return patches, w0, ho


def kernel(x, w0, b0, w_dw1, b_dw1, w_proj1, b_proj1, w_exp2, b_exp2, w_dw2, b_dw2, w_proj2, b_proj2, w_exp3, b_exp3, w_dw3, b_dw3, w_proj3, b_proj3, w_exp4, b_exp4, w_dw4, b_dw4, w_proj4, b_proj4, w_exp5, b_exp5, w_dw5, b_dw5, w_proj5, b_proj5, w_exp6, b_exp6, w_dw6, b_dw6, w_proj6, b_proj6, w_exp7, b_exp7, w_dw7, b_dw7, w_proj7, b_proj7, w_exp8, b_exp8, w_dw8, b_dw8, w_proj8, b_proj8, w_exp9, b_exp9, w_dw9, b_dw9, w_proj9, b_proj9, w_exp10, b_exp10, w_dw10, b_dw10, w_proj10, b_proj10, w_exp11, b_exp11, w_dw11, b_dw11, w_proj11, b_proj11, w_exp12, b_exp12, w_dw12, b_dw12, w_proj12, b_proj12, w_exp13, b_exp13, w_dw13, b_dw13, w_proj13, b_proj13, w_exp14, b_exp14, w_dw14, b_dw14, w_proj14, b_proj14, w_exp15, b_exp15, w_dw15, b_dw15, w_proj15, b_proj15, w_dw16, b_dw16, w_proj16, b_proj16, w_exp17, b_exp17, w_dw17, b_dw17, w_proj17, b_proj17, w_exp18, b_exp18, w_dw18, b_dw18, w_proj18, b_proj18, w_exp19, b_exp19, w_dw19, b_dw19, w_proj19, b_proj19, w_exp20, b_exp20, w_dw20, b_dw20, w_proj20, b_proj20, w_exp21, b_exp21, w_dw21, b_dw21, w_proj21, b_proj21, w_exp22, b_exp22, w_dw22, b_dw22, w_proj22, b_proj22, w23, b23):
    lp = {
        1: (w_dw1, b_dw1, w_proj1, b_proj1),
        2: (w_exp2, b_exp2, w_dw2, b_dw2, w_proj2, b_proj2),
        3: (w_exp3, b_exp3, w_dw3, b_dw3, w_proj3, b_proj3),
        4: (w_exp4, b_exp4, w_dw4, b_dw4, w_proj4, b_proj4),
        5: (w_exp5, b_exp5, w_dw5, b_dw5, w_proj5, b_proj5),
        6: (w_exp6, b_exp6, w_dw6, b_dw6, w_proj6, b_proj6),
        7: (w_exp7, b_exp7, w_dw7, b_dw7, w_proj7, b_proj7),
        8: (w_exp8, b_exp8, w_dw8, b_dw8, w_proj8, b_proj8),
        9: (w_exp9, b_exp9, w_dw9, b_dw9, w_proj9, b_proj9),
        10: (w_exp10, b_exp10, w_dw10, b_dw10, w_proj10, b_proj10),
        11: (w_exp11, b_exp11, w_dw11, b_dw11, w_proj11, b_proj11),
        12: (w_exp12, b_exp12, w_dw12, b_dw12, w_proj12, b_proj12),
        13: (w_exp13, b_exp13, w_dw13, b_dw13, w_proj13, b_proj13),
        14: (w_exp14, b_exp14, w_dw14, b_dw14, w_proj14, b_proj14),
        15: (w_exp15, b_exp15, w_dw15, b_dw15, w_proj15, b_proj15),
        16: (w_dw16, b_dw16, w_proj16, b_proj16),
        17: (w_exp17, b_exp17, w_dw17, b_dw17, w_proj17, b_proj17),
        18: (w_exp18, b_exp18, w_dw18, b_dw18, w_proj18, b_proj18),
        19: (w_exp19, b_exp19, w_dw19, b_dw19, w_proj19, b_proj19),
        20: (w_exp20, b_exp20, w_dw20, b_dw20, w_proj20, b_proj20),
        21: (w_exp21, b_exp21, w_dw21, b_dw21, w_proj21, b_proj21),
        22: (w_exp22, b_exp22, w_dw22, b_dw22, w_proj22, b_proj22),
    }
    patches, w36, h_stem = _stem_patches(x, w0)
    geoms = _build_geoms(h_stem)
    nb = x.shape[0]

    cur = _stage_call(patches, [geoms[1]], [lp[1]], stem_wb=(w36, b0),
                      pitch_wsp=_PITCHES[0])

    outs = []
    for stage, pitch in zip(_STAGES[1:], _PITCHES[1:]):
        g0, gl = geoms[stage[0]], geoms[stage[-1]]
        x2d = cur.reshape(nb, cur.shape[1] // 2, 2 * cur.shape[2])
        head = lp[stage[0]]
        w_e, b_e = head[0], head[1]
        zw = jnp.zeros_like(w_e)
        w_bd = jnp.concatenate(
            [jnp.concatenate([w_e, zw], axis=1),
             jnp.concatenate([zw, w_e], axis=1)], axis=0)
        b_t = jnp.concatenate([b_e, b_e], axis=1)
        stage_params = [(w_bd, b_t) + tuple(head[2:])] + [lp[i] for i in stage[1:]]
        cur = _stage_call(x2d, [geoms[i] for i in stage], stage_params,
                          pitch_wsp=pitch)
        out = cur.reshape(nb, gl["ws"], gl["ws"], gl["cout"])
        outs.append(jnp.transpose(out, (0, 3, 1, 2)).astype(jnp.float32))
    return outs
